# baseline (speedup 1.0000x reference)
import functools

import jax
import jax.numpy as jnp
from jax.experimental import pallas as pl
from jax.experimental.pallas import tpu as pltpu
from jax.experimental.pallas import tpu_sc as plsc

DIM = 64
SEQ = 200
WINDOW = 128
K_WIN = 4


def _hash_body(t_ref, o_ref):
    t = t_ref[...]
    n, l = t.shape
    zcol = jnp.zeros((n, 1), jnp.int32)
    left = jnp.concatenate([zcol, t[:, :-1]], axis=1)
    carry = jnp.concatenate([jnp.zeros((1, 1), jnp.int32), t[:-1, -1:]], axis=0)
    lane = jax.lax.broadcasted_iota(jnp.int32, (n, l), 1)
    row = jax.lax.broadcasted_iota(jnp.int32, (n, l), 0)
    prev = jnp.where(lane == 0, carry, left)
    first = ((row * l + lane) % SEQ) == 0
    prev = jnp.where(first, t, prev)
    o_ref[...] = prev * 3 + t


def _bigram_idx(tok2d):
    return pl.pallas_call(
        _hash_body,
        out_shape=jax.ShapeDtypeStruct(tok2d.shape, jnp.int32),
    )(tok2d)


def _sc_gather(table, idx2d):
    n_windows = idx2d.shape[0]
    n_rows = n_windows * WINDOW
    n_workers = 32
    wpw = n_windows // n_workers
    n_chunks = wpw // K_WIN
    chunk_rows = K_WIN * WINDOW
    mesh = plsc.VectorSubcoreMesh(core_axis_name="c", subcore_axis_name="s")

    @functools.partial(
        pl.kernel,
        out_type=jax.ShapeDtypeStruct((n_rows, DIM), jnp.float32),
        mesh=mesh,
        compiler_params=pltpu.CompilerParams(
            use_tc_tiling_on_sc=False,
            skip_device_barrier=True,
            disable_bounds_checks=True,
            disable_semaphore_checks=True,
        ),
        scratch_types=[
            pltpu.VMEM((wpw, WINDOW), jnp.int32),
            pltpu.VMEM((chunk_rows, DIM), jnp.float32),
            pltpu.VMEM((chunk_rows, DIM), jnp.float32),
            pltpu.SemaphoreType.DMA,
            pltpu.SemaphoreType.DMA,
            pltpu.SemaphoreType.DMA,
            pltpu.SemaphoreType.DMA,
        ],
    )
    def k(table_hbm, idx_hbm, out_hbm, idx_all, buf_a, buf_b, gs_a, gs_b, ss_a, ss_b):
        i32 = jnp.int32
        wid = jax.lax.axis_index("s") * i32(2) + jax.lax.axis_index("c")
        w0 = wid.astype(i32) * i32(wpw)

        pltpu.sync_copy(idx_hbm.at[pl.ds(w0, wpw)], idx_all)

        def gathers(c, buf, sem):
            copies = []
            for j in range(K_WIN):
                row = c * i32(K_WIN) + i32(j)
                copies.append(pltpu.make_async_copy(
                    table_hbm.at[idx_all.at[row]],
                    buf.at[pl.ds(j * WINDOW, WINDOW)],
                    sem,
                ))
            return copies

        def fire(copies):
            for cp in copies:
                cp.start()

        def drain(copies):
            for cp in copies:
                cp.wait()

        def store(c, buf, sem):
            base = (w0 + c * i32(K_WIN)) * i32(WINDOW)
            return pltpu.make_async_copy(
                buf, out_hbm.at[pl.ds(base, chunk_rows)], sem)

        fire(gathers(i32(0), buf_a, gs_a))
        fire(gathers(i32(1), buf_b, gs_b))

        @pl.loop(0, n_chunks, step=2)
        def _(c0):
            c = c0.astype(i32)

            drain(gathers(c, buf_a, gs_a))
            st_a = store(c, buf_a, ss_a)
            st_a.start()
            st_a.wait()

            @pl.when(c + i32(2) < i32(n_chunks))
            def _():
                fire(gathers(c + i32(2), buf_a, gs_a))

            drain(gathers(c + i32(1), buf_b, gs_b))
            st_b = store(c + i32(1), buf_b, ss_b)
            st_b.start()
            st_b.wait()

            @pl.when(c + i32(3) < i32(n_chunks))
            def _():
                fire(gathers(c + i32(3), buf_b, gs_b))

    return k(table, idx2d)


def kernel(token_ids, emb_weight):
    b, s = token_ids.shape
    with jax._src.config.enable_x64(False):
        tok2d = token_ids.astype(jnp.int32).reshape(b * s // WINDOW, WINDOW)
        idx2d = _bigram_idx(tok2d)
        out = _sc_gather(emb_weight, idx2d)
        return out.reshape(b, s, DIM)


# speedup vs baseline: 1.0007x; 1.0007x over previous
import functools

import jax
import jax.numpy as jnp
from jax.experimental import pallas as pl
from jax.experimental.pallas import tpu as pltpu
from jax.experimental.pallas import tpu_sc as plsc

DIM = 64
SEQ = 200
WINDOW = 128
K_WIN = 4


def _hash_body(t_ref, o_ref):
    t = t_ref[...]
    n, l = t.shape
    zcol = jnp.zeros((n, 1), jnp.int32)
    left = jnp.concatenate([zcol, t[:, :-1]], axis=1)
    carry = jnp.concatenate([jnp.zeros((1, 1), jnp.int32), t[:-1, -1:]], axis=0)
    lane = jax.lax.broadcasted_iota(jnp.int32, (n, l), 1)
    row = jax.lax.broadcasted_iota(jnp.int32, (n, l), 0)
    prev = jnp.where(lane == 0, carry, left)
    first = ((row * l + lane) % SEQ) == 0
    prev = jnp.where(first, t, prev)
    o_ref[...] = prev * 3 + t


def _bigram_idx(tok2d):
    return pl.pallas_call(
        _hash_body,
        out_shape=jax.ShapeDtypeStruct(tok2d.shape, jnp.int32),
    )(tok2d)


def _sc_gather(table_flat, idx_flat):
    n_rows = idx_flat.shape[0]
    n_windows = n_rows // WINDOW
    n_workers = 32
    wpw = n_windows // n_workers
    n_chunks = wpw // K_WIN
    chunk_rows = K_WIN * WINDOW
    mesh = plsc.VectorSubcoreMesh(core_axis_name="c", subcore_axis_name="s")

    @functools.partial(
        pl.kernel,
        out_type=jax.ShapeDtypeStruct((n_rows, DIM), jnp.float32),
        mesh=mesh,
        compiler_params=pltpu.CompilerParams(
            use_tc_tiling_on_sc=False,
            skip_device_barrier=True,
            disable_bounds_checks=True,
            disable_semaphore_checks=True,
        ),
        scratch_types=[
            pltpu.VMEM((wpw * WINDOW,), jnp.int32),
            pltpu.VMEM((chunk_rows, DIM), jnp.float32),
            pltpu.VMEM((chunk_rows, DIM), jnp.float32),
            pltpu.SemaphoreType.DMA,
            pltpu.SemaphoreType.DMA,
            pltpu.SemaphoreType.DMA,
            pltpu.SemaphoreType.DMA,
        ],
    )
    def k(table_hbm, idx_hbm, out_hbm, idx_all, buf_a, buf_b, gs_a, gs_b, ss_a, ss_b):
        i32 = jnp.int32
        table2d = table_hbm
        wid = jax.lax.axis_index("s") * i32(2) + jax.lax.axis_index("c")
        base0 = wid.astype(i32) * i32(wpw * WINDOW)

        pltpu.sync_copy(idx_hbm.at[pl.ds(base0, wpw * WINDOW)], idx_all)

        def gathers(c, buf, sem):
            copies = []
            for j in range(K_WIN):
                off = (c * i32(K_WIN) + i32(j)) * i32(WINDOW)
                copies.append(pltpu.make_async_copy(
                    table2d.at[idx_all.at[pl.ds(off, WINDOW)]],
                    buf.at[pl.ds(j * WINDOW, WINDOW)],
                    sem,
                ))
            return copies

        def fire(copies):
            for cp in copies:
                cp.start()

        def drain(copies):
            for cp in copies:
                cp.wait()

        def store(c, buf, sem):
            base = base0 + c * i32(chunk_rows)
            return pltpu.make_async_copy(
                buf, out_hbm.at[pl.ds(base, chunk_rows)], sem)

        fire(gathers(i32(0), buf_a, gs_a))
        fire(gathers(i32(1), buf_b, gs_b))

        @pl.loop(0, n_chunks, step=2)
        def _(c0):
            c = c0.astype(i32)

            drain(gathers(c, buf_a, gs_a))
            st_a = store(c, buf_a, ss_a)
            st_a.start()
            st_a.wait()

            @pl.when(c + i32(2) < i32(n_chunks))
            def _():
                fire(gathers(c + i32(2), buf_a, gs_a))

            drain(gathers(c + i32(1), buf_b, gs_b))
            st_b = store(c + i32(1), buf_b, ss_b)
            st_b.start()
            st_b.wait()

            @pl.when(c + i32(3) < i32(n_chunks))
            def _():
                fire(gathers(c + i32(3), buf_b, gs_b))

    return k(table_flat, idx_flat)


def kernel(token_ids, emb_weight):
    b, s = token_ids.shape
    with jax._src.config.enable_x64(False):
        tok2d = token_ids.astype(jnp.int32).reshape(b * s // WINDOW, WINDOW)
        idx2d = _bigram_idx(tok2d)
        out = _sc_gather(emb_weight, idx2d.reshape(-1))
        return out.reshape(b, s, DIM)


# speedup vs baseline: 1.9363x; 1.9350x over previous
import functools

import jax
import jax.numpy as jnp
from jax.experimental import pallas as pl
from jax.experimental.pallas import tpu as pltpu
from jax.experimental.pallas import tpu_sc as plsc

DIM = 64
SEQ = 200
WINDOW = 128
K_WIN = 4


def _hash_body(t_ref, o_ref):
    t = t_ref[...]
    n, l = t.shape
    zcol = jnp.zeros((n, 1), jnp.int32)
    left = jnp.concatenate([zcol, t[:, :-1]], axis=1)
    carry = jnp.concatenate([jnp.zeros((1, 1), jnp.int32), t[:-1, -1:]], axis=0)
    lane = jax.lax.broadcasted_iota(jnp.int32, (n, l), 1)
    row = jax.lax.broadcasted_iota(jnp.int32, (n, l), 0)
    prev = jnp.where(lane == 0, carry, left)
    first = ((row * l + lane) % SEQ) == 0
    prev = jnp.where(first, t, prev)
    o_ref[...] = prev * 3 + t


def _bigram_idx(tok2d):
    return pl.pallas_call(
        _hash_body,
        out_shape=jax.ShapeDtypeStruct(tok2d.shape, jnp.int32),
    )(tok2d)


def _sc_gather(table_flat, idx_flat):
    n_rows = idx_flat.shape[0]
    n_windows = n_rows // WINDOW
    n_workers = 32
    wpw = n_windows // n_workers
    n_chunks = wpw // K_WIN
    chunk_rows = K_WIN * WINDOW
    mesh = plsc.VectorSubcoreMesh(core_axis_name="c", subcore_axis_name="s")

    @functools.partial(
        pl.kernel,
        out_type=jax.ShapeDtypeStruct((n_rows, 2 * DIM), jnp.float32),
        mesh=mesh,
        compiler_params=pltpu.CompilerParams(
            use_tc_tiling_on_sc=False,
            skip_device_barrier=True,
            disable_bounds_checks=True,
            disable_semaphore_checks=True,
        ),
        scratch_types=[
            pltpu.VMEM((wpw * WINDOW,), jnp.int32),
            pltpu.VMEM((chunk_rows, DIM), jnp.float32),
            pltpu.VMEM((chunk_rows, DIM), jnp.float32),
            pltpu.SemaphoreType.DMA,
            pltpu.SemaphoreType.DMA,
            pltpu.SemaphoreType.DMA,
            pltpu.SemaphoreType.DMA,
        ],
    )
    def k(table_hbm, idx_hbm, out_hbm, idx_all, buf_a, buf_b, gs_a, gs_b, ss_a, ss_b):
        i32 = jnp.int32
        table2d = table_hbm
        wid = jax.lax.axis_index("s") * i32(2) + jax.lax.axis_index("c")
        base0 = wid.astype(i32) * i32(wpw * WINDOW)

        pltpu.sync_copy(idx_hbm.at[pl.ds(base0, wpw * WINDOW)], idx_all)

        def gathers(c, buf, sem):
            copies = []
            for j in range(K_WIN):
                off = (c * i32(K_WIN) + i32(j)) * i32(WINDOW)
                copies.append(pltpu.make_async_copy(
                    table2d.at[idx_all.at[pl.ds(off, WINDOW)]],
                    buf.at[pl.ds(j * WINDOW, WINDOW)],
                    sem,
                ))
            return copies

        def fire(copies):
            for cp in copies:
                cp.start()

        def drain(copies):
            for cp in copies:
                cp.wait()

        def store(c, buf, sem):
            base = base0 + c * i32(chunk_rows)
            return pltpu.make_async_copy(
                buf, out_hbm.at[pl.ds(base, chunk_rows), pl.ds(0, DIM)], sem)

        fire(gathers(i32(0), buf_a, gs_a))
        fire(gathers(i32(1), buf_b, gs_b))

        @pl.loop(0, n_chunks, step=2)
        def _(c0):
            c = c0.astype(i32)

            drain(gathers(c, buf_a, gs_a))
            st_a = store(c, buf_a, ss_a)
            st_a.start()
            st_a.wait()

            @pl.when(c + i32(2) < i32(n_chunks))
            def _():
                fire(gathers(c + i32(2), buf_a, gs_a))

            drain(gathers(c + i32(1), buf_b, gs_b))
            st_b = store(c + i32(1), buf_b, ss_b)
            st_b.start()
            st_b.wait()

            @pl.when(c + i32(3) < i32(n_chunks))
            def _():
                fire(gathers(c + i32(3), buf_b, gs_b))

    return k(table_flat, idx_flat)


def kernel(token_ids, emb_weight):
    b, s = token_ids.shape
    with jax._src.config.enable_x64(False):
        tok2d = token_ids.astype(jnp.int32).reshape(b * s // WINDOW, WINDOW)
        idx2d = _bigram_idx(tok2d)
        out = _sc_gather(emb_weight[:400000], idx2d.reshape(-1))
        return out[:, :DIM].reshape(b, s, DIM)


# speedup vs baseline: 1.9386x; 1.0012x over previous
import functools

import jax
import jax.numpy as jnp
from jax.experimental import pallas as pl
from jax.experimental.pallas import tpu as pltpu
from jax.experimental.pallas import tpu_sc as plsc

DIM = 64
SEQ = 200
WINDOW = 256
K_WIN = 2


def _hash_body(t_ref, o_ref):
    t = t_ref[...]
    n, l = t.shape
    zcol = jnp.zeros((n, 1), jnp.int32)
    left = jnp.concatenate([zcol, t[:, :-1]], axis=1)
    carry = jnp.concatenate([jnp.zeros((1, 1), jnp.int32), t[:-1, -1:]], axis=0)
    lane = jax.lax.broadcasted_iota(jnp.int32, (n, l), 1)
    row = jax.lax.broadcasted_iota(jnp.int32, (n, l), 0)
    prev = jnp.where(lane == 0, carry, left)
    first = ((row * l + lane) % SEQ) == 0
    prev = jnp.where(first, t, prev)
    o_ref[...] = prev * 3 + t


def _bigram_idx(tok2d):
    return pl.pallas_call(
        _hash_body,
        out_shape=jax.ShapeDtypeStruct(tok2d.shape, jnp.int32),
    )(tok2d)


def _sc_gather(table_flat, idx_flat):
    n_rows = idx_flat.shape[0]
    n_windows = n_rows // WINDOW
    n_workers = 32
    wpw = n_windows // n_workers
    n_chunks = wpw // K_WIN
    chunk_rows = K_WIN * WINDOW
    mesh = plsc.VectorSubcoreMesh(core_axis_name="c", subcore_axis_name="s")

    @functools.partial(
        pl.kernel,
        out_type=jax.ShapeDtypeStruct((n_rows, 2 * DIM), jnp.float32),
        mesh=mesh,
        compiler_params=pltpu.CompilerParams(
            use_tc_tiling_on_sc=False,
            skip_device_barrier=True,
            disable_bounds_checks=True,
            disable_semaphore_checks=True,
        ),
        scratch_types=[
            pltpu.VMEM((wpw * WINDOW,), jnp.int32),
            pltpu.VMEM((chunk_rows, DIM), jnp.float32),
            pltpu.VMEM((chunk_rows, DIM), jnp.float32),
            pltpu.SemaphoreType.DMA,
            pltpu.SemaphoreType.DMA,
            pltpu.SemaphoreType.DMA,
            pltpu.SemaphoreType.DMA,
        ],
    )
    def k(table_hbm, idx_hbm, out_hbm, idx_all, buf_a, buf_b, gs_a, gs_b, ss_a, ss_b):
        i32 = jnp.int32
        table2d = table_hbm
        wid = jax.lax.axis_index("s") * i32(2) + jax.lax.axis_index("c")
        base0 = wid.astype(i32) * i32(wpw * WINDOW)

        pltpu.sync_copy(idx_hbm.at[pl.ds(base0, wpw * WINDOW)], idx_all)

        def gathers(c, buf, sem):
            copies = []
            for j in range(K_WIN):
                off = (c * i32(K_WIN) + i32(j)) * i32(WINDOW)
                copies.append(pltpu.make_async_copy(
                    table2d.at[idx_all.at[pl.ds(off, WINDOW)]],
                    buf.at[pl.ds(j * WINDOW, WINDOW)],
                    sem,
                ))
            return copies

        def fire(copies):
            for cp in copies:
                cp.start()

        def drain(copies):
            for cp in copies:
                cp.wait()

        def store(c, buf, sem):
            base = base0 + c * i32(chunk_rows)
            return pltpu.make_async_copy(
                buf, out_hbm.at[pl.ds(base, chunk_rows), pl.ds(0, DIM)], sem)

        fire(gathers(i32(0), buf_a, gs_a))
        fire(gathers(i32(1), buf_b, gs_b))

        @pl.loop(0, n_chunks, step=2)
        def _(c0):
            c = c0.astype(i32)

            drain(gathers(c, buf_a, gs_a))
            st_a = store(c, buf_a, ss_a)
            st_a.start()
            st_a.wait()

            @pl.when(c + i32(2) < i32(n_chunks))
            def _():
                fire(gathers(c + i32(2), buf_a, gs_a))

            drain(gathers(c + i32(1), buf_b, gs_b))
            st_b = store(c + i32(1), buf_b, ss_b)
            st_b.start()
            st_b.wait()

            @pl.when(c + i32(3) < i32(n_chunks))
            def _():
                fire(gathers(c + i32(3), buf_b, gs_b))

    return k(table_flat, idx_flat)


def kernel(token_ids, emb_weight):
    b, s = token_ids.shape
    with jax._src.config.enable_x64(False):
        tok2d = token_ids.astype(jnp.int32).reshape(b * s // WINDOW, WINDOW)
        idx2d = _bigram_idx(tok2d)
        out = _sc_gather(emb_weight[:400000], idx2d.reshape(-1))
        return out[:, :DIM].reshape(b, s, DIM)
